# baseline (speedup 1.0000x reference)
import functools

import jax
import jax.numpy as jnp
from jax import lax
from jax.experimental import pallas as pl
from jax.experimental.pallas import tpu as pltpu, tpu_sc as plsc

_NUM_CORES = 2
_NUM_SUBCORES = 16
_NUM_WORKERS = _NUM_CORES * _NUM_SUBCORES


@functools.cache
def _sc_copy(n):
    chunk = n // _NUM_WORKERS
    mesh = plsc.VectorSubcoreMesh(core_axis_name="c", subcore_axis_name="s")

    @functools.partial(
        pl.kernel,
        mesh=mesh,
        out_type=(
            jax.ShapeDtypeStruct((n,), jnp.float32),
            jax.ShapeDtypeStruct((n,), jnp.float32),
        ),
        scratch_types=[
            pltpu.SemaphoreType.DMA,
            pltpu.SemaphoreType.DMA,
        ],
    )
    def body(kx_hbm, vx_hbm, k_out_hbm, v_out_hbm, ksem, vsem):
        wid = lax.axis_index("s") * _NUM_CORES + lax.axis_index("c")
        base = wid * chunk
        sl = pl.ds(base, chunk)
        kcp = pltpu.make_async_copy(kx_hbm.at[sl], k_out_hbm.at[sl], ksem)
        vcp = pltpu.make_async_copy(vx_hbm.at[sl], v_out_hbm.at[sl], vsem)
        kcp.start()
        vcp.start()
        kcp.wait()
        vcp.wait()

    return body


def kernel(kx, vx, k_cache, v_cache):
    del k_cache, v_cache
    shape = kx.shape
    n = kx.size
    k_flat, v_flat = _sc_copy(n)(kx.reshape(n), vx.reshape(n))
    return k_flat.reshape(shape), v_flat.reshape(shape)


# speedup vs baseline: 1.7038x; 1.7038x over previous
import functools

import jax
import jax.numpy as jnp
from jax import lax
from jax.experimental import pallas as pl
from jax.experimental.pallas import tpu as pltpu, tpu_sc as plsc

_NUM_CORES = 2
_NUM_SUBCORES = 16
_NUM_WORKERS = _NUM_CORES * _NUM_SUBCORES


@functools.cache
def _sc_copy(n):
    chunk = n // _NUM_WORKERS
    mesh = plsc.VectorSubcoreMesh(core_axis_name="c", subcore_axis_name="s")

    @functools.partial(
        pl.kernel,
        mesh=mesh,
        out_type=(
            jax.ShapeDtypeStruct((n,), jnp.float32),
            jax.ShapeDtypeStruct((n,), jnp.float32),
        ),
        scratch_types=[
            pltpu.VMEM((chunk,), jnp.float32),
            pltpu.VMEM((chunk,), jnp.float32),
            pltpu.SemaphoreType.DMA,
            pltpu.SemaphoreType.DMA,
        ],
    )
    def body(kx_hbm, vx_hbm, k_out_hbm, v_out_hbm, kbuf, vbuf, ksem, vsem):
        wid = lax.axis_index("s") * _NUM_CORES + lax.axis_index("c")
        base = wid * chunk
        sl = pl.ds(base, chunk)
        kld = pltpu.make_async_copy(kx_hbm.at[sl], kbuf, ksem)
        vld = pltpu.make_async_copy(vx_hbm.at[sl], vbuf, vsem)
        kld.start()
        vld.start()
        kld.wait()
        kst = pltpu.make_async_copy(kbuf, k_out_hbm.at[sl], ksem)
        kst.start()
        vld.wait()
        vst = pltpu.make_async_copy(vbuf, v_out_hbm.at[sl], vsem)
        vst.start()
        kst.wait()
        vst.wait()

    return body


def kernel(kx, vx, k_cache, v_cache):
    del k_cache, v_cache
    shape = kx.shape
    n = kx.size
    k_flat, v_flat = _sc_copy(n)(kx.reshape(n), vx.reshape(n))
    return k_flat.reshape(shape), v_flat.reshape(shape)


# speedup vs baseline: 1.8169x; 1.0663x over previous
import functools

import jax
import jax.numpy as jnp
from jax import lax
from jax.experimental import pallas as pl
from jax.experimental.pallas import tpu as pltpu, tpu_sc as plsc

_NUM_CORES = 2
_NUM_SUBCORES = 16
_NUM_WORKERS = _NUM_CORES * _NUM_SUBCORES


@functools.cache
def _sc_copy(n):
    mesh = plsc.ScalarSubcoreMesh(axis_name="c", num_cores=_NUM_CORES)

    @functools.partial(
        pl.kernel,
        mesh=mesh,
        out_type=(
            jax.ShapeDtypeStruct((n,), jnp.float32),
            jax.ShapeDtypeStruct((n,), jnp.float32),
        ),
        scratch_types=[
            pltpu.VMEM_SHARED((n,), jnp.float32),
            pltpu.SemaphoreType.DMA,
        ],
    )
    def body(kx_hbm, vx_hbm, k_out_hbm, v_out_hbm, buf, sem):
        c = lax.axis_index("c")

        @pl.when(c == 0)
        def _copy_k():
            pltpu.make_async_copy(kx_hbm, buf, sem).start()
            pltpu.make_async_copy(kx_hbm, buf, sem).wait()
            pltpu.make_async_copy(buf, k_out_hbm, sem).start()
            pltpu.make_async_copy(buf, k_out_hbm, sem).wait()

        @pl.when(c == 1)
        def _copy_v():
            pltpu.make_async_copy(vx_hbm, buf, sem).start()
            pltpu.make_async_copy(vx_hbm, buf, sem).wait()
            pltpu.make_async_copy(buf, v_out_hbm, sem).start()
            pltpu.make_async_copy(buf, v_out_hbm, sem).wait()

    return body


def kernel(kx, vx, k_cache, v_cache):
    del k_cache, v_cache
    shape = kx.shape
    n = kx.size
    k_flat, v_flat = _sc_copy(n)(kx.reshape(n), vx.reshape(n))
    return k_flat.reshape(shape), v_flat.reshape(shape)


# speedup vs baseline: 1.8192x; 1.0013x over previous
import functools

import jax
import jax.numpy as jnp
from jax import lax
from jax.experimental import pallas as pl
from jax.experimental.pallas import tpu as pltpu, tpu_sc as plsc

_NUM_CORES = 2


@functools.cache
def _sc_copy(n):
    half = n // _NUM_CORES
    mesh = plsc.ScalarSubcoreMesh(axis_name="c", num_cores=_NUM_CORES)

    @functools.partial(
        pl.kernel,
        mesh=mesh,
        out_type=(
            jax.ShapeDtypeStruct((n,), jnp.float32),
            jax.ShapeDtypeStruct((n,), jnp.float32),
        ),
        scratch_types=[
            pltpu.VMEM_SHARED((half,), jnp.float32),
            pltpu.VMEM_SHARED((half,), jnp.float32),
            pltpu.SemaphoreType.DMA,
            pltpu.SemaphoreType.DMA,
        ],
    )
    def body(kx_hbm, vx_hbm, k_out_hbm, v_out_hbm, kbuf, vbuf, ksem, vsem):
        c = lax.axis_index("c")
        sl = pl.ds(c * half, half)
        kld = pltpu.make_async_copy(kx_hbm.at[sl], kbuf, ksem)
        vld = pltpu.make_async_copy(vx_hbm.at[sl], vbuf, vsem)
        kld.start()
        vld.start()
        kld.wait()
        kst = pltpu.make_async_copy(kbuf, k_out_hbm.at[sl], ksem)
        kst.start()
        vld.wait()
        vst = pltpu.make_async_copy(vbuf, v_out_hbm.at[sl], vsem)
        vst.start()
        kst.wait()
        vst.wait()

    return body


def kernel(kx, vx, k_cache, v_cache):
    del k_cache, v_cache
    shape = kx.shape
    n = kx.size
    k_flat, v_flat = _sc_copy(n)(kx.reshape(n), vx.reshape(n))
    return k_flat.reshape(shape), v_flat.reshape(shape)


# speedup vs baseline: 1.9048x; 1.0471x over previous
import functools

import jax
import jax.numpy as jnp
from jax import lax
from jax.experimental import pallas as pl
from jax.experimental.pallas import tpu as pltpu, tpu_sc as plsc

_NUM_CORES = 2


@functools.cache
def _sc_copy(n):
    mesh = plsc.ScalarSubcoreMesh(axis_name="c", num_cores=1)

    @functools.partial(
        pl.kernel,
        mesh=mesh,
        out_type=(
            jax.ShapeDtypeStruct((n,), jnp.float32),
            jax.ShapeDtypeStruct((n,), jnp.float32),
        ),
        scratch_types=[
            pltpu.VMEM_SHARED((n,), jnp.float32),
            pltpu.VMEM_SHARED((n,), jnp.float32),
            pltpu.SemaphoreType.DMA,
            pltpu.SemaphoreType.DMA,
        ],
    )
    def body(kx_hbm, vx_hbm, k_out_hbm, v_out_hbm, kbuf, vbuf, ksem, vsem):
        kld = pltpu.make_async_copy(kx_hbm, kbuf, ksem)
        vld = pltpu.make_async_copy(vx_hbm, vbuf, vsem)
        kld.start()
        vld.start()
        kld.wait()
        kst = pltpu.make_async_copy(kbuf, k_out_hbm, ksem)
        kst.start()
        vld.wait()
        vst = pltpu.make_async_copy(vbuf, v_out_hbm, vsem)
        vst.start()
        kst.wait()
        vst.wait()

    return body


def kernel(kx, vx, k_cache, v_cache):
    del k_cache, v_cache
    shape = kx.shape
    n = kx.size
    k_flat, v_flat = _sc_copy(n)(kx.reshape(n), vx.reshape(n))
    return k_flat.reshape(shape), v_flat.reshape(shape)
